# baseline (speedup 1.0000x reference)
import jax
import jax.numpy as jnp
from jax import lax
from jax.experimental import pallas as pl
from jax.experimental.pallas import tpu as pltpu
from jax.experimental.pallas import tpu_sc as plsc

_E = 8
_T = 4096
_S = 24576
_NW = 32
_WT = _S // _NW
_L = 16
_NEG_INF = float("-inf")


def _logits_body(x_ref, w_ref, b_ref, out_ref):
    out_ref[...] = lax.dot_general(
        w_ref[...], x_ref[...],
        (((1,), (1,)), ((), ())),
        preferred_element_type=jnp.float32,
    ) + b_ref[...]


def _fused_body(x_ref, w_ref, b_ref, gates_ref, idx_ref):
    logits = lax.dot_general(
        w_ref[...], x_ref[...],
        (((1,), (1,)), ((), ())),
        preferred_element_type=jnp.float32,
    ) + b_ref[...]

    m1 = jnp.max(logits, axis=0, keepdims=True)
    i1 = jnp.full((1, _T), _E - 1, dtype=jnp.int32)
    for e in range(_E - 2, -1, -1):
        i1 = jnp.where(logits[e:e + 1, :] == m1, jnp.int32(e), i1)

    eiota = lax.broadcasted_iota(jnp.int32, (_E, _T), 0)
    masked = jnp.where(eiota == i1, _NEG_INF, logits)
    m2 = jnp.max(masked, axis=0, keepdims=True)
    i2 = jnp.full((1, _T), _E - 1, dtype=jnp.int32)
    for e in range(_E - 2, -1, -1):
        i2 = jnp.where(masked[e:e + 1, :] == m2, jnp.int32(e), i2)

    e2 = jnp.exp(m2 - m1)
    r = 1.0 / (1.0 + e2)
    g1 = r
    g2 = e2 * r

    gates_ref[...] = jnp.where(eiota == i1, g1, jnp.where(eiota == i2, g2, 0.0))
    idx_ref[...] = jnp.concatenate([i1, i2], axis=0)


def _route_body(logits_hbm, gates_hbm, idx_hbm, lg_v, g_v, i_v):
    wid = lax.axis_index("s") * 2 + lax.axis_index("c")
    t0 = wid * _WT
    pltpu.sync_copy(logits_hbm.at[:, pl.ds(t0, _WT)], lg_v)

    def chunk(ci, _):
        base = ci * _L
        v = [lg_v[e, pl.ds(base, _L)] for e in range(_E)]

        m1 = v[0]
        for e in range(1, _E):
            m1 = jnp.maximum(m1, v[e])
        i1 = jnp.full((_L,), _E - 1, dtype=jnp.int32)
        for e in range(_E - 2, -1, -1):
            i1 = jnp.where(v[e] == m1, jnp.int32(e), i1)

        vm = [jnp.where(i1 == e, _NEG_INF, v[e]) for e in range(_E)]
        m2 = vm[0]
        for e in range(1, _E):
            m2 = jnp.maximum(m2, vm[e])
        i2 = jnp.full((_L,), _E - 1, dtype=jnp.int32)
        for e in range(_E - 2, -1, -1):
            i2 = jnp.where(vm[e] == m2, jnp.int32(e), i2)

        e2 = jnp.exp(m2 - m1)
        r = 1.0 / (1.0 + e2)
        g1 = r
        g2 = e2 * r

        zero = jnp.zeros((_L,), jnp.float32)
        for e in range(_E):
            g_v[e, pl.ds(base, _L)] = jnp.where(
                i1 == e, g1, jnp.where(i2 == e, g2, zero))
        i_v[0, pl.ds(base, _L)] = i1
        i_v[1, pl.ds(base, _L)] = i2
        return 0

    lax.fori_loop(0, _WT // _L, chunk, 0)

    pltpu.sync_copy(g_v, gates_hbm.at[:, pl.ds(t0, _WT)])
    pltpu.sync_copy(i_v, idx_hbm.at[:, pl.ds(t0, _WT)])


def _route(logits_t):
    mesh = plsc.VectorSubcoreMesh(
        core_axis_name="c", subcore_axis_name="s",
        num_cores=2, num_subcores=16)
    return pl.kernel(
        _route_body,
        out_type=[
            jax.ShapeDtypeStruct((_E, _S), jnp.float32),
            jax.ShapeDtypeStruct((2, _S), jnp.int32),
        ],
        mesh=mesh,
        scratch_types=[
            pltpu.VMEM((_E, _WT), jnp.float32),
            pltpu.VMEM((_E, _WT), jnp.float32),
            pltpu.VMEM((2, _WT), jnp.int32),
        ],
        compiler_params=pltpu.CompilerParams(
            needs_layout_passes=False, use_tc_tiling_on_sc=True),
    )(logits_t)


def _merge_body(ga_ref, gb_ref, ia_ref, ib_ref, gates_ref, idx_ref):
    g = jnp.concatenate([ga_ref[...], gb_ref[...]], axis=1)
    i = jnp.concatenate([ia_ref[...], ib_ref[...]], axis=1)
    tokens = g.shape[1]
    gates_ref[...] = g.T.reshape(gates_ref.shape)
    idx_ref[...] = i.T.reshape(idx_ref.shape)


def kernel(x, W, b):
    B, N, D = x.shape
    tokens = B * N
    x2 = x.reshape(tokens, D)
    b2 = b.reshape(_E, 1)
    grid_a = _S // _T
    grid_b = (tokens - _S) // _T

    logits_a = pl.pallas_call(
        _logits_body,
        grid=(grid_a,),
        in_specs=[
            pl.BlockSpec((_T, D), lambda i: (i, 0)),
            pl.BlockSpec((_E, D), lambda i: (0, 0)),
            pl.BlockSpec((_E, 1), lambda i: (0, 0)),
        ],
        out_specs=pl.BlockSpec((_E, _T), lambda i: (0, i)),
        out_shape=jax.ShapeDtypeStruct((_E, _S), jnp.float32),
    )(x2, W, b2)

    gates_a, idx_a = _route(logits_a)

    gates_b, idx_b = pl.pallas_call(
        _fused_body,
        grid=(grid_b,),
        in_specs=[
            pl.BlockSpec((_T, D), lambda i: (i + grid_a, 0)),
            pl.BlockSpec((_E, D), lambda i: (0, 0)),
            pl.BlockSpec((_E, 1), lambda i: (0, 0)),
        ],
        out_specs=[
            pl.BlockSpec((_E, _T), lambda i: (0, i)),
            pl.BlockSpec((2, _T), lambda i: (0, i)),
        ],
        out_shape=[
            jax.ShapeDtypeStruct((_E, tokens - _S), jnp.float32),
            jax.ShapeDtypeStruct((2, tokens - _S), jnp.int32),
        ],
    )(x2, W, b2)

    full_gates, topk_idx = pl.pallas_call(
        _merge_body,
        out_shape=[
            jax.ShapeDtypeStruct((B, N, _E), jnp.float32),
            jax.ShapeDtypeStruct((B, N, 2), jnp.int32),
        ],
    )(gates_a, gates_b, idx_a, idx_b)
    return (full_gates, topk_idx)


# speedup vs baseline: 1.6197x; 1.6197x over previous
import jax
import jax.numpy as jnp
from jax import lax
from jax.experimental import pallas as pl
from jax.experimental.pallas import tpu as pltpu
from jax.experimental.pallas import tpu_sc as plsc

_E = 8
_T = 4096
_S = 24576
_NW = 32
_WT = _S // _NW
_L = 16
_NEG_INF = float("-inf")


def _logits_body(x_ref, w_ref, b_ref, out_ref):
    out_ref[...] = lax.dot_general(
        w_ref[...], x_ref[...],
        (((1,), (1,)), ((), ())),
        preferred_element_type=jnp.float32,
    ) + b_ref[...]


def _fused_body(x_ref, w_ref, b_ref, gates_ref, idx_ref):
    logits = lax.dot_general(
        w_ref[...], x_ref[...],
        (((1,), (1,)), ((), ())),
        preferred_element_type=jnp.float32,
    ) + b_ref[...]

    m1 = jnp.max(logits, axis=0, keepdims=True)
    i1 = jnp.full((1, _T), _E - 1, dtype=jnp.int32)
    for e in range(_E - 2, -1, -1):
        i1 = jnp.where(logits[e:e + 1, :] == m1, jnp.int32(e), i1)

    eiota = lax.broadcasted_iota(jnp.int32, (_E, _T), 0)
    masked = jnp.where(eiota == i1, _NEG_INF, logits)
    m2 = jnp.max(masked, axis=0, keepdims=True)
    i2 = jnp.full((1, _T), _E - 1, dtype=jnp.int32)
    for e in range(_E - 2, -1, -1):
        i2 = jnp.where(masked[e:e + 1, :] == m2, jnp.int32(e), i2)

    e2 = jnp.exp(m2 - m1)
    r = 1.0 / (1.0 + e2)
    g1 = r
    g2 = e2 * r

    gates_ref[...] = jnp.where(eiota == i1, g1, jnp.where(eiota == i2, g2, 0.0))
    idx_ref[...] = jnp.concatenate([i1, i2], axis=0)


def _route_body(logits_hbm, gates_hbm, idx_hbm, lg_v, g_v, i_v):
    wid = lax.axis_index("s") * 2 + lax.axis_index("c")
    t0 = wid * _WT
    pltpu.sync_copy(logits_hbm.at[:, pl.ds(t0, _WT)], lg_v)

    def chunk(ci, _):
        base = ci * _L
        v = [lg_v[e, pl.ds(base, _L)] for e in range(_E)]

        m1 = v[0]
        for e in range(1, _E):
            m1 = jnp.maximum(m1, v[e])
        i1 = jnp.full((_L,), _E - 1, dtype=jnp.int32)
        for e in range(_E - 2, -1, -1):
            i1 = jnp.where(v[e] == m1, jnp.int32(e), i1)

        vm = [jnp.where(i1 == e, _NEG_INF, v[e]) for e in range(_E)]
        m2 = vm[0]
        for e in range(1, _E):
            m2 = jnp.maximum(m2, vm[e])
        i2 = jnp.full((_L,), _E - 1, dtype=jnp.int32)
        for e in range(_E - 2, -1, -1):
            i2 = jnp.where(vm[e] == m2, jnp.int32(e), i2)

        e2 = jnp.exp(m2 - m1)
        r = 1.0 / (1.0 + e2)
        g1 = r
        g2 = e2 * r

        zero = jnp.zeros((_L,), jnp.float32)
        for e in range(_E):
            g_v[e, pl.ds(base, _L)] = jnp.where(
                i1 == e, g1, jnp.where(i2 == e, g2, zero))
        i_v[0, pl.ds(base, _L)] = i1
        i_v[1, pl.ds(base, _L)] = i2
        return 0

    lax.fori_loop(0, _WT // _L, chunk, 0)

    pltpu.sync_copy(g_v, gates_hbm.at[:, pl.ds(t0, _WT)])
    pltpu.sync_copy(i_v, idx_hbm.at[:, pl.ds(t0, _WT)])


def _route(logits_t):
    mesh = plsc.VectorSubcoreMesh(
        core_axis_name="c", subcore_axis_name="s",
        num_cores=2, num_subcores=16)
    return pl.kernel(
        _route_body,
        out_type=[
            jax.ShapeDtypeStruct((_E, _S), jnp.float32),
            jax.ShapeDtypeStruct((2, _S), jnp.int32),
        ],
        mesh=mesh,
        scratch_types=[
            pltpu.VMEM((_E, _WT), jnp.float32),
            pltpu.VMEM((_E, _WT), jnp.float32),
            pltpu.VMEM((2, _WT), jnp.int32),
        ],
        compiler_params=pltpu.CompilerParams(
            needs_layout_passes=False, use_tc_tiling_on_sc=True),
    )(logits_t)


def kernel(x, W, b):
    B, N, D = x.shape
    tokens = B * N
    x2 = x.reshape(tokens, D)
    b2 = b.reshape(_E, 1)
    grid_a = _S // _T
    grid_b = (tokens - _S) // _T

    logits_a = pl.pallas_call(
        _logits_body,
        grid=(grid_a,),
        in_specs=[
            pl.BlockSpec((_T, D), lambda i: (i, 0)),
            pl.BlockSpec((_E, D), lambda i: (0, 0)),
            pl.BlockSpec((_E, 1), lambda i: (0, 0)),
        ],
        out_specs=pl.BlockSpec((_E, _T), lambda i: (0, i)),
        out_shape=jax.ShapeDtypeStruct((_E, _S), jnp.float32),
    )(x2, W, b2)

    gates_a, idx_a = _route(logits_a)

    gates_b, idx_b = pl.pallas_call(
        _fused_body,
        grid=(grid_b,),
        in_specs=[
            pl.BlockSpec((_T, D), lambda i: (i + grid_a, 0)),
            pl.BlockSpec((_E, D), lambda i: (0, 0)),
            pl.BlockSpec((_E, 1), lambda i: (0, 0)),
        ],
        out_specs=[
            pl.BlockSpec((_E, _T), lambda i: (0, i)),
            pl.BlockSpec((2, _T), lambda i: (0, i)),
        ],
        out_shape=[
            jax.ShapeDtypeStruct((_E, tokens - _S), jnp.float32),
            jax.ShapeDtypeStruct((2, tokens - _S), jnp.int32),
        ],
    )(x2, W, b2)

    full_gates = jnp.concatenate([gates_a, gates_b], axis=1).T.reshape(B, N, _E)
    topk_idx = jnp.concatenate([idx_a, idx_b], axis=1).T.reshape(B, N, 2)
    return (full_gates, topk_idx)


# speedup vs baseline: 1.6339x; 1.0087x over previous
import jax
import jax.numpy as jnp
from jax import lax
from jax.experimental import pallas as pl
from jax.experimental.pallas import tpu as pltpu
from jax.experimental.pallas import tpu_sc as plsc

_E = 8
_T = 4096
_S = 32768
_NW = 32
_WT = _S // _NW
_L = 16
_NEG_INF = float("-inf")


def _logits_body(x_ref, w_ref, b_ref, out_ref):
    out_ref[...] = lax.dot_general(
        w_ref[...], x_ref[...],
        (((1,), (1,)), ((), ())),
        preferred_element_type=jnp.float32,
    ) + b_ref[...]


def _fused_body(x_ref, w_ref, b_ref, gates_ref, idx_ref):
    logits = lax.dot_general(
        w_ref[...], x_ref[...],
        (((1,), (1,)), ((), ())),
        preferred_element_type=jnp.float32,
    ) + b_ref[...]

    m1 = jnp.max(logits, axis=0, keepdims=True)
    i1 = jnp.full((1, _T), _E - 1, dtype=jnp.int32)
    for e in range(_E - 2, -1, -1):
        i1 = jnp.where(logits[e:e + 1, :] == m1, jnp.int32(e), i1)

    eiota = lax.broadcasted_iota(jnp.int32, (_E, _T), 0)
    masked = jnp.where(eiota == i1, _NEG_INF, logits)
    m2 = jnp.max(masked, axis=0, keepdims=True)
    i2 = jnp.full((1, _T), _E - 1, dtype=jnp.int32)
    for e in range(_E - 2, -1, -1):
        i2 = jnp.where(masked[e:e + 1, :] == m2, jnp.int32(e), i2)

    e2 = jnp.exp(m2 - m1)
    r = 1.0 / (1.0 + e2)
    g1 = r
    g2 = e2 * r

    gates_ref[...] = jnp.where(eiota == i1, g1, jnp.where(eiota == i2, g2, 0.0))
    idx_ref[...] = jnp.concatenate([i1, i2], axis=0)


def _route_body(logits_hbm, gates_hbm, idx_hbm, lg_v, g_v, i_v):
    wid = lax.axis_index("s") * 2 + lax.axis_index("c")
    t0 = wid * _WT
    pltpu.sync_copy(logits_hbm.at[:, pl.ds(t0, _WT)], lg_v)

    def chunk(ci, _):
        base = ci * _L
        v = [lg_v[e, pl.ds(base, _L)] for e in range(_E)]

        m1 = v[0]
        for e in range(1, _E):
            m1 = jnp.maximum(m1, v[e])
        i1 = jnp.full((_L,), _E - 1, dtype=jnp.int32)
        for e in range(_E - 2, -1, -1):
            i1 = jnp.where(v[e] == m1, jnp.int32(e), i1)

        vm = [jnp.where(i1 == e, _NEG_INF, v[e]) for e in range(_E)]
        m2 = vm[0]
        for e in range(1, _E):
            m2 = jnp.maximum(m2, vm[e])
        i2 = jnp.full((_L,), _E - 1, dtype=jnp.int32)
        for e in range(_E - 2, -1, -1):
            i2 = jnp.where(vm[e] == m2, jnp.int32(e), i2)

        e2 = jnp.exp(m2 - m1)
        r = 1.0 / (1.0 + e2)
        g1 = r
        g2 = e2 * r

        zero = jnp.zeros((_L,), jnp.float32)
        for e in range(_E):
            g_v[e, pl.ds(base, _L)] = jnp.where(
                i1 == e, g1, jnp.where(i2 == e, g2, zero))
        i_v[0, pl.ds(base, _L)] = i1
        i_v[1, pl.ds(base, _L)] = i2
        return 0

    lax.fori_loop(0, _WT // _L, chunk, 0)

    pltpu.sync_copy(g_v, gates_hbm.at[:, pl.ds(t0, _WT)])
    pltpu.sync_copy(i_v, idx_hbm.at[:, pl.ds(t0, _WT)])


def _route(logits_t):
    mesh = plsc.VectorSubcoreMesh(
        core_axis_name="c", subcore_axis_name="s",
        num_cores=2, num_subcores=16)
    return pl.kernel(
        _route_body,
        out_type=[
            jax.ShapeDtypeStruct((_E, _S), jnp.float32),
            jax.ShapeDtypeStruct((2, _S), jnp.int32),
        ],
        mesh=mesh,
        scratch_types=[
            pltpu.VMEM((_E, _WT), jnp.float32),
            pltpu.VMEM((_E, _WT), jnp.float32),
            pltpu.VMEM((2, _WT), jnp.int32),
        ],
        compiler_params=pltpu.CompilerParams(
            needs_layout_passes=False, use_tc_tiling_on_sc=True),
    )(logits_t)


def kernel(x, W, b):
    B, N, D = x.shape
    tokens = B * N
    x2 = x.reshape(tokens, D)
    b2 = b.reshape(_E, 1)
    grid_a = _S // _T

    logits_a = pl.pallas_call(
        _logits_body,
        grid=(grid_a,),
        in_specs=[
            pl.BlockSpec((_T, D), lambda i: (i, 0)),
            pl.BlockSpec((_E, D), lambda i: (0, 0)),
            pl.BlockSpec((_E, 1), lambda i: (0, 0)),
        ],
        out_specs=pl.BlockSpec((_E, _T), lambda i: (0, i)),
        out_shape=jax.ShapeDtypeStruct((_E, _S), jnp.float32),
    )(x2, W, b2)

    gates_a, idx_a = _route(logits_a)

    full_gates = gates_a.T.reshape(B, N, _E)
    topk_idx = idx_a.T.reshape(B, N, 2)
    return (full_gates, topk_idx)


# speedup vs baseline: 1.6796x; 1.0280x over previous
import jax
import jax.numpy as jnp
from jax import lax
from jax.experimental import pallas as pl
from jax.experimental.pallas import tpu as pltpu
from jax.experimental.pallas import tpu_sc as plsc

_E = 8
_T = 4096
_S = 32768
_NW = 32
_WT = _S // _NW
_L = 16
_NEG_INF = float("-inf")


def _logits_body(x_ref, w_ref, b_ref, out_ref):
    out_ref[...] = lax.dot_general(
        w_ref[...], x_ref[...],
        (((1,), (1,)), ((), ())),
        preferred_element_type=jnp.float32,
    ) + b_ref[...]


def _route_body(logits_hbm, gates_hbm, idx_hbm, lg_v, g_v, i_v):
    wid = lax.axis_index("s") * 2 + lax.axis_index("c")
    t0 = wid * _WT
    pltpu.sync_copy(logits_hbm.at[:, pl.ds(t0, _WT)], lg_v)

    def chunk(ci, _):
        base = ci * _L
        v = [lg_v[e, pl.ds(base, _L)] for e in range(_E)]

        m1 = v[0]
        for e in range(1, _E):
            m1 = jnp.maximum(m1, v[e])
        i1 = jnp.full((_L,), _E - 1, dtype=jnp.int32)
        for e in range(_E - 2, -1, -1):
            i1 = jnp.where(v[e] == m1, jnp.int32(e), i1)

        vm = [jnp.where(i1 == e, _NEG_INF, v[e]) for e in range(_E)]
        m2 = vm[0]
        for e in range(1, _E):
            m2 = jnp.maximum(m2, vm[e])
        i2 = jnp.full((_L,), _E - 1, dtype=jnp.int32)
        for e in range(_E - 2, -1, -1):
            i2 = jnp.where(vm[e] == m2, jnp.int32(e), i2)

        e2 = jnp.exp(m2 - m1)
        r = 1.0 / (1.0 + e2)
        g1 = r
        g2 = e2 * r

        zero = jnp.zeros((_L,), jnp.float32)
        for e in range(_E):
            g_v[e, pl.ds(base, _L)] = jnp.where(
                i1 == e, g1, jnp.where(i2 == e, g2, zero))
        i_v[0, pl.ds(base, _L)] = i1
        i_v[1, pl.ds(base, _L)] = i2
        return 0

    lax.fori_loop(0, _WT // _L, chunk, 0)

    pltpu.sync_copy(g_v, gates_hbm.at[:, pl.ds(t0, _WT)])
    pltpu.sync_copy(i_v, idx_hbm.at[:, pl.ds(t0, _WT)])


def _route(logits_t):
    mesh = plsc.VectorSubcoreMesh(
        core_axis_name="c", subcore_axis_name="s",
        num_cores=2, num_subcores=16)
    return pl.kernel(
        _route_body,
        out_type=[
            jax.ShapeDtypeStruct((_E, _S), jnp.float32),
            jax.ShapeDtypeStruct((2, _S), jnp.int32),
        ],
        mesh=mesh,
        scratch_types=[
            pltpu.VMEM((_E, _WT), jnp.float32),
            pltpu.VMEM((_E, _WT), jnp.float32),
            pltpu.VMEM((2, _WT), jnp.int32),
        ],
        compiler_params=pltpu.CompilerParams(
            needs_layout_passes=False, use_tc_tiling_on_sc=True,
            disable_bounds_checks=True),
    )(logits_t)


def kernel(x, W, b):
    B, N, D = x.shape
    tokens = B * N
    x2 = x.reshape(tokens, D)
    b2 = b.reshape(_E, 1)
    grid_a = _S // _T

    logits_a = pl.pallas_call(
        _logits_body,
        grid=(grid_a,),
        in_specs=[
            pl.BlockSpec((_T, D), lambda i: (i, 0)),
            pl.BlockSpec((_E, D), lambda i: (0, 0)),
            pl.BlockSpec((_E, 1), lambda i: (0, 0)),
        ],
        out_specs=pl.BlockSpec((_E, _T), lambda i: (0, i)),
        out_shape=jax.ShapeDtypeStruct((_E, _S), jnp.float32),
    )(x2, W, b2)

    gates_a, idx_a = _route(logits_a)

    full_gates = gates_a.T.reshape(B, N, _E)
    topk_idx = idx_a.T.reshape(B, N, 2)
    return (full_gates, topk_idx)
